# Initial kernel scaffold; baseline (speedup 1.0000x reference)
import jax
import jax.numpy as jnp
from jax.experimental import pallas as pl


def kernel(logits, val_freqs):
    raise NotImplementedError("write your pallas kernel here")


# speedup vs baseline: 1190.0441x; 1190.0441x over previous
import jax
import jax.numpy as jnp
import numpy as np
from jax.experimental import pallas as pl
from jax.experimental.pallas import tpu as pltpu

_NUM_BINS = 15
_NUM_CLASSES = 4
_NW = 3
_SMOOTH = 1e-8
_H = 512
_W = 512


def _body(x_ref, t_ref, o_ref):
    f32 = jnp.float32
    xs = [x_ref[0, c] for c in range(_NUM_CLASSES)]
    m = jnp.maximum(jnp.maximum(xs[0], xs[1]), jnp.maximum(xs[2], xs[3]))
    es = [jnp.exp(x - m) for x in xs]
    s = (es[0] + es[1]) + (es[2] + es[3])
    ps = [e / s for e in es]
    p0, p1, p2, p3 = ps
    bm = [
        (p0 >= p1) & (p0 >= p2) & (p0 >= p3),
        (p1 > p0) & (p1 >= p2) & (p1 >= p3),
        (p2 > p0) & (p2 > p1) & (p2 >= p3),
        (p3 > p0) & (p3 > p1) & (p3 > p2),
    ]
    width = np.float32(1.0 / _NUM_BINS)
    zc = jnp.zeros((_H, 1), f32)
    zr = jnp.zeros((1, _W), f32)
    vals = []
    for c in range(_NUM_CLASSES):
        bf = jnp.where(bm[c], f32(1.0), f32(0.0))
        rs = bf + jnp.concatenate([bf[:, 1:], zc], axis=1) \
                + jnp.concatenate([zc, bf[:, :-1]], axis=1)
        cs = rs + jnp.concatenate([rs[1:, :], zr], axis=0) \
                + jnp.concatenate([zr, rs[:-1, :]], axis=0)
        cnt = (cs - bf).astype(jnp.int32)
        bin_i = jnp.clip(jnp.floor(ps[c] / width).astype(jnp.int32), 0,
                         _NUM_BINS - 1)
        code = cnt * _NUM_BINS + bin_i
        tlo = jnp.broadcast_to(t_ref[c, 0], (_H, 128))
        thi = jnp.broadcast_to(t_ref[c, 1], (_H, 128))
        ilo = jnp.minimum(code, 127)
        ihi = jnp.clip(code - 128, 0, 127)
        glo = jnp.take_along_axis(
            tlo, ilo, axis=-1, mode=jax.lax.GatherScatterMode.PROMISE_IN_BOUNDS)
        ghi = jnp.take_along_axis(
            thi, ihi, axis=-1, mode=jax.lax.GatherScatterMode.PROMISE_IN_BOUNDS)
        vals.append(jnp.where(code < 128, glo, ghi))
    sv = (vals[0] + vals[1]) + (vals[2] + vals[3])
    sv = jnp.where(sv == 0.0, f32(_SMOOTH), sv)
    for c in range(_NUM_CLASSES):
        o_ref[0, c] = vals[c] / sv


def kernel(logits, val_freqs):
    B = logits.shape[0]
    tflat = val_freqs.reshape(_NUM_CLASSES, _NW * _NW * _NUM_BINS)
    table = jnp.zeros((_NUM_CLASSES, 2, 128), jnp.float32)
    table = table.at[:, 0, :].set(tflat[:, :128])
    table = table.at[:, 1, : _NW * _NW * _NUM_BINS - 128].set(tflat[:, 128:])
    return pl.pallas_call(
        _body,
        grid=(B,),
        in_specs=[
            pl.BlockSpec((1, _NUM_CLASSES, _H, _W), lambda i: (i, 0, 0, 0)),
            pl.BlockSpec((_NUM_CLASSES, 2, 128), lambda i: (0, 0, 0)),
        ],
        out_specs=pl.BlockSpec((1, _NUM_CLASSES, _H, _W),
                               lambda i: (i, 0, 0, 0)),
        out_shape=jax.ShapeDtypeStruct(logits.shape, jnp.float32),
    )(logits, table)
